# baseline (speedup 1.0000x reference)
import functools

import jax
import jax.numpy as jnp
from jax import lax
from jax.experimental import pallas as pl
from jax.experimental.pallas import tpu as pltpu
from jax.experimental.pallas import tpu_sc as plsc

_L = 16
_NC = 2
_NS = 16


def _rsqrt_newton(s):
    bits = plsc.bitcast(s, jnp.int32)
    y = plsc.bitcast(jnp.full((_L,), 0x5F3759DF, jnp.int32) - (bits >> 1),
                     jnp.float32)
    for _ in range(3):
        y = y * (1.5 - 0.5 * s * y * y)
    return y


@functools.cache
def _make_stats(e, n_pad):
    ch = 2000
    per_tile = e // _NS
    n_chunks = per_tile // ch
    groups = ch // _L
    rpt = n_pad // _NS
    mesh = plsc.VectorSubcoreMesh(core_axis_name="c", subcore_axis_name="s",
                                  num_cores=_NC, num_subcores=_NS)

    @functools.partial(
        pl.kernel,
        out_type=jax.ShapeDtypeStruct((_NC, n_pad), jnp.float32),
        mesh=mesh,
        scratch_types=[
            pltpu.VMEM((ch,), jnp.int32),
            pltpu.VMEM((ch,), jnp.float32),
            pltpu.VMEM((n_pad,), jnp.float32),
            pltpu.VMEM((_NS, rpt), jnp.float32),
            pltpu.VMEM((rpt,), jnp.float32),
            pltpu.VMEM_SHARED((_NS, n_pad), jnp.float32),
        ],
        compiler_params=pltpu.CompilerParams(needs_layout_passes=False),
    )
    def stats(ids2, attrs, r_out, ids_v, at_v, table, colblk, rbuf, staging):
        c = lax.axis_index("c")
        s = lax.axis_index("s")

        def zero_body(i, _):
            table[pl.ds(i * _L, _L)] = jnp.zeros((_L,), jnp.float32)
            return 0
        lax.fori_loop(0, n_pad // _L, zero_body, 0)

        ebase = s * per_tile
        for chunk in range(n_chunks):
            base = ebase + chunk * ch
            pltpu.sync_copy(ids2.at[pl.ds(c * e + base, ch)], ids_v)
            pltpu.sync_copy(attrs.at[pl.ds(base, ch)], at_v)

            def upd(g, _):
                sl = pl.ds(g * _L, _L)
                plsc.addupdate_scatter(table, [ids_v[sl]], jnp.exp(at_v[sl]))
                return 0
            lax.fori_loop(0, groups, upd, 0)

        pltpu.sync_copy(table, staging.at[s])
        plsc.subcore_barrier()
        pltpu.sync_copy(staging.at[:, pl.ds(s * rpt, rpt)], colblk)

        def red(g, _):
            sl = pl.ds(g * _L, _L)
            acc = colblk[0, sl]
            for j in range(1, _NS):
                acc = acc + colblk[j, sl]
            rbuf[sl] = _rsqrt_newton(acc)
            return 0
        lax.fori_loop(0, rpt // _L, red, 0)
        pltpu.sync_copy(rbuf, r_out.at[c, pl.ds(s * rpt, rpt)])

    return stats


@functools.cache
def _make_msg(n, e, n_pad, dh):
    k = 80
    ring = 3
    per_tile = e // _NS
    n_chunks = per_tile // k
    g_per_k = k // _L
    fch = dh // _L
    rows_out = n_pad // _NS
    n_ob = rows_out // k
    mesh = plsc.VectorSubcoreMesh(core_axis_name="c", subcore_axis_name="s",
                                  num_cores=_NC, num_subcores=_NS)

    idx_t = [pltpu.VMEM((k,), jnp.int32) for _ in range(ring)]
    val_t = [pltpu.VMEM((k,), jnp.float32) for _ in range(ring)]
    row_t = [pltpu.VMEM((k, dh), jnp.float32) for _ in range(ring)]

    @functools.partial(
        pl.kernel,
        out_type=jax.ShapeDtypeStruct((_NC, n_pad, dh), jnp.float32),
        mesh=mesh,
        scratch_types=(
            [pltpu.VMEM((n_pad,), jnp.float32)]
            + [pltpu.VMEM((rows_out,), jnp.float32)]
            + idx_t + idx_t + idx_t
            + val_t + val_t
            + row_t
            + [
                pltpu.VMEM_SHARED((n_pad, dh), jnp.float32),
                pltpu.SemaphoreType.DMA,
                pltpu.SemaphoreType.DMA,
                pltpu.SemaphoreType.DMA,
            ]
        ),
        compiler_params=pltpu.CompilerParams(needs_layout_passes=False),
    )
    def msg(x0, x1, ids2, attrs, r2, out,
            rout_t, rin_ep,
            sidx0, sidx1, sidx2, didx0, didx1, didx2, dsc0, dsc1, dsc2,
            ab0, ab1, ab2, wb0, wb1, wb2, rw0, rw1, rw2,
            acc, gsem, ssem, isem):
        sidx = [sidx0, sidx1, sidx2]
        didx = [didx0, didx1, didx2]
        dscat = [dsc0, dsc1, dsc2]
        abuf = [ab0, ab1, ab2]
        wbuf = [wb0, wb1, wb2]
        rows = [rw0, rw1, rw2]
        c = lax.axis_index("c")
        s = lax.axis_index("s")
        pltpu.sync_copy(r2.at[1], rout_t)
        pltpu.sync_copy(r2.at[0, pl.ds(s * rows_out, rows_out)], rin_ep)

        def zb(i, _):
            for j in range(fch):
                rw0[i, pl.ds(j * _L, _L)] = jnp.zeros((_L,), jnp.float32)
            return 0
        lax.fori_loop(0, k, zb, 0)
        for t in range(n_ob):
            pltpu.sync_copy(rw0, acc.at[pl.ds(s * rows_out + t * k, k)])
        plsc.subcore_barrier()

        ebase = s * per_tile

        def issue_idx(ci, b):
            base = ebase + ci * k
            pltpu.async_copy(ids2.at[pl.ds(base, k)], didx[b], isem)
            pltpu.async_copy(ids2.at[pl.ds(e + base, k)], sidx[b], isem)
            pltpu.async_copy(attrs.at[pl.ds(base, k)], abuf[b], isem)

        def wait_idx(b):
            for r in (didx[b], sidx[b]):
                pltpu.make_async_copy(ids2.at[pl.ds(0, k)], r, isem).wait()
            pltpu.make_async_copy(attrs.at[pl.ds(0, k)], abuf[b], isem).wait()

        def compute_w(b):
            for g in range(g_per_k):
                sl = pl.ds(g * _L, _L)
                ro = plsc.load_gather(rout_t, [sidx[b][sl]])
                wbuf[b][sl] = jnp.exp(abuf[b][sl]) * ro

        def copy_dscat(b):
            for g in range(g_per_k):
                sl = pl.ds(g * _L, _L)
                dscat[b][sl] = didx[b][sl]

        def issue_gather(b):
            @pl.when(c == 0)
            def _():
                pltpu.async_copy(x0.at[sidx[b]], rows[b], gsem)

            @pl.when(c == 1)
            def _():
                pltpu.async_copy(x1.at[sidx[b]], rows[b], gsem)

        def wait_gather(b):
            pltpu.make_async_copy(x0.at[pl.ds(0, k)], rows[b], gsem).wait()

        def issue_scatter(b):
            pltpu.async_copy(rows[b], acc.at[dscat[b]], ssem, add=True)

        def drain_scatter(b):
            pltpu.make_async_copy(x0.at[pl.ds(0, k)], rows[b], ssem).wait()

        def scale(b):
            rw = rows[b]
            wr = wbuf[b]

            @plsc.parallel_loop(0, g_per_k, unroll=1)
            def body(g):
                wv = wr[pl.ds(g * _L, _L)]
                for t in range(_L):
                    w16 = jnp.take_along_axis(
                        wv, jnp.full((_L,), t, jnp.int32), axis=0,
                        mode="promise_in_bounds")
                    i = g * _L + t
                    for fj in range(fch):
                        sl = pl.ds(fj * _L, _L)
                        rw[i, sl] = rw[i, sl] * w16

        def step(ci, b, drain):
            nb = (b + 1) % ring
            b2 = (b + 2) % ring
            wait_idx(nb)
            compute_w(nb)
            if drain:
                drain_scatter(nb)
            copy_dscat(nb)
            issue_gather(nb)
            wait_gather(b)
            issue_idx(ci + 2, b2)
            scale(b)
            issue_scatter(b)

        issue_idx(jnp.int32(0), 0)
        wait_idx(0)
        compute_w(0)
        copy_dscat(0)
        issue_gather(0)
        issue_idx(jnp.int32(1), 1)
        step(jnp.int32(0), 0, drain=False)
        step(jnp.int32(1), 1, drain=False)

        def tri(j, _):
            ci = 2 + 3 * j
            step(ci, 2, drain=True)
            step(ci + 1, 0, drain=True)
            step(ci + 2, 1, drain=True)
            return 0
        lax.fori_loop(0, (n_chunks - 2) // 3, tri, 0)
        drain_scatter(0)
        drain_scatter(1)
        wait_gather(2)
        wait_idx(0)

        plsc.subcore_barrier()
        for t in range(n_ob):
            rb = s * rows_out + t * k
            pltpu.sync_copy(acc.at[pl.ds(rb, k)], rw0)

            @plsc.parallel_loop(0, g_per_k, unroll=1)
            def ob(g):
                rv = rin_ep[pl.ds(t * k + g * _L, _L)]
                for u in range(_L):
                    r16 = jnp.take_along_axis(
                        rv, jnp.full((_L,), u, jnp.int32), axis=0,
                        mode="promise_in_bounds")
                    i = g * _L + u
                    for fj in range(fch):
                        sl = pl.ds(fj * _L, _L)
                        rw0[i, sl] = rw0[i, sl] * r16
            pltpu.sync_copy(rw0, out.at[c, pl.ds(rb, k)])

    return msg


def kernel(x, edge_index, edge_attrs):
    n, d = x.shape
    e = edge_index.shape[1]
    dh = d // 2
    n_pad = ((n + 255) // 256) * 256
    ids2 = jnp.concatenate(
        [edge_index[1], edge_index[0], jnp.zeros((256,), jnp.int32)])
    attrs = jnp.concatenate([edge_attrs, jnp.zeros((256,), jnp.float32)])
    r2 = _make_stats(e, n_pad)(ids2, attrs)
    o = _make_msg(n, e, n_pad, dh)(x[:, :dh], x[:, dh:], ids2, attrs, r2)
    return jnp.concatenate([o[0, :n], o[1, :n]], axis=1)


# speedup vs baseline: 1.1898x; 1.1898x over previous
import functools

import jax
import jax.numpy as jnp
from jax import lax
from jax.experimental import pallas as pl
from jax.experimental.pallas import tpu as pltpu
from jax.experimental.pallas import tpu_sc as plsc

_L = 16
_NC = 2
_NS = 16


def _rsqrt_newton(s):
    bits = plsc.bitcast(s, jnp.int32)
    y = plsc.bitcast(jnp.full((_L,), 0x5F3759DF, jnp.int32) - (bits >> 1),
                     jnp.float32)
    for _ in range(3):
        y = y * (1.5 - 0.5 * s * y * y)
    return y


@functools.cache
def _make_stats(e, n_pad):
    ch = 2000
    per_tile = e // _NS
    n_chunks = per_tile // ch
    groups = ch // _L
    rpt = n_pad // _NS
    mesh = plsc.VectorSubcoreMesh(core_axis_name="c", subcore_axis_name="s",
                                  num_cores=_NC, num_subcores=_NS)

    @functools.partial(
        pl.kernel,
        out_type=jax.ShapeDtypeStruct((_NC, n_pad), jnp.float32),
        mesh=mesh,
        scratch_types=[
            pltpu.VMEM((ch,), jnp.int32),
            pltpu.VMEM((ch,), jnp.float32),
            pltpu.VMEM((n_pad,), jnp.float32),
            pltpu.VMEM((_NS, rpt), jnp.float32),
            pltpu.VMEM((rpt,), jnp.float32),
            pltpu.VMEM_SHARED((_NS, n_pad), jnp.float32),
        ],
        compiler_params=pltpu.CompilerParams(needs_layout_passes=False),
    )
    def stats(ids2, attrs, r_out, ids_v, at_v, table, colblk, rbuf, staging):
        c = lax.axis_index("c")
        s = lax.axis_index("s")

        def zero_body(i, _):
            table[pl.ds(i * _L, _L)] = jnp.zeros((_L,), jnp.float32)
            return 0
        lax.fori_loop(0, n_pad // _L, zero_body, 0)

        ebase = s * per_tile
        for chunk in range(n_chunks):
            base = ebase + chunk * ch
            pltpu.sync_copy(ids2.at[pl.ds(c * e + base, ch)], ids_v)
            pltpu.sync_copy(attrs.at[pl.ds(base, ch)], at_v)

            def upd(g, _):
                sl = pl.ds(g * _L, _L)
                plsc.addupdate_scatter(table, [ids_v[sl]], jnp.exp(at_v[sl]))
                return 0
            lax.fori_loop(0, groups, upd, 0)

        pltpu.sync_copy(table, staging.at[s])
        plsc.subcore_barrier()
        pltpu.sync_copy(staging.at[:, pl.ds(s * rpt, rpt)], colblk)

        def red(g, _):
            sl = pl.ds(g * _L, _L)
            acc = colblk[0, sl]
            for j in range(1, _NS):
                acc = acc + colblk[j, sl]
            rbuf[sl] = _rsqrt_newton(acc)
            return 0
        lax.fori_loop(0, rpt // _L, red, 0)
        pltpu.sync_copy(rbuf, r_out.at[c, pl.ds(s * rpt, rpt)])

    return stats


@functools.cache
def _make_msg(n, e, n_pad, dh):
    k = 80
    ring = 3
    per_tile = e // _NS
    n_chunks = per_tile // k
    g_per_k = k // _L
    fch = dh // _L
    rows_out = n_pad // _NS
    n_ob = rows_out // k
    mesh = plsc.VectorSubcoreMesh(core_axis_name="c", subcore_axis_name="s",
                                  num_cores=_NC, num_subcores=_NS)

    pk_t = [pltpu.VMEM((3 * k,), jnp.int32) for _ in range(ring)]
    idx_t = [pltpu.VMEM((k,), jnp.int32) for _ in range(ring)]
    val_t = [pltpu.VMEM((k,), jnp.float32) for _ in range(ring)]
    row_t = [pltpu.VMEM((k, dh), jnp.float32) for _ in range(ring)]

    @functools.partial(
        pl.kernel,
        out_type=jax.ShapeDtypeStruct((_NC, n_pad, dh), jnp.float32),
        mesh=mesh,
        scratch_types=(
            [pltpu.VMEM((n_pad,), jnp.float32)]
            + [pltpu.VMEM((rows_out,), jnp.float32)]
            + pk_t
            + idx_t
            + val_t
            + row_t
            + [
                pltpu.VMEM_SHARED((n_pad, dh), jnp.float32),
                pltpu.SemaphoreType.DMA,
                pltpu.SemaphoreType.DMA,
                pltpu.SemaphoreType.DMA,
            ]
        ),
        compiler_params=pltpu.CompilerParams(needs_layout_passes=False),
    )
    def msg(x0, x1, pk, r2, out,
            rout_t, rin_ep,
            pk0, pk1, pk2, dsc0, dsc1, dsc2, wb0, wb1, wb2, rw0, rw1, rw2,
            acc, gsem, ssem, isem):
        pbuf = [pk0, pk1, pk2]
        dscat = [dsc0, dsc1, dsc2]
        wbuf = [wb0, wb1, wb2]
        rows = [rw0, rw1, rw2]
        c = lax.axis_index("c")
        s = lax.axis_index("s")
        pltpu.sync_copy(r2.at[1], rout_t)
        pltpu.sync_copy(r2.at[0, pl.ds(s * rows_out, rows_out)], rin_ep)

        def zb(i, _):
            for j in range(fch):
                rw0[i, pl.ds(j * _L, _L)] = jnp.zeros((_L,), jnp.float32)
            return 0
        lax.fori_loop(0, k, zb, 0)
        for t in range(n_ob):
            pltpu.sync_copy(rw0, acc.at[pl.ds(s * rows_out + t * k, k)])
        plsc.subcore_barrier()

        def issue_idx(ci, b):
            base = (s * n_chunks + ci) * (3 * k)
            pltpu.async_copy(pk.at[pl.ds(base, 3 * k)], pbuf[b], isem)

        def wait_idx(b):
            pltpu.make_async_copy(pk.at[pl.ds(0, 3 * k)], pbuf[b], isem).wait()

        def compute_w(b):
            for g in range(g_per_k):
                sg = pbuf[b][pl.ds(k + g * _L, _L)]
                av = plsc.bitcast(pbuf[b][pl.ds(2 * k + g * _L, _L)],
                                  jnp.float32)
                ro = plsc.load_gather(rout_t, [sg])
                wbuf[b][pl.ds(g * _L, _L)] = jnp.exp(av) * ro

        def copy_dscat(b):
            for g in range(g_per_k):
                sl = pl.ds(g * _L, _L)
                dscat[b][sl] = pbuf[b][sl]

        def issue_gather(b):
            idxr = pbuf[b].at[pl.ds(k, k)]

            @pl.when(c == 0)
            def _():
                pltpu.async_copy(x0.at[idxr], rows[b], gsem)

            @pl.when(c == 1)
            def _():
                pltpu.async_copy(x1.at[idxr], rows[b], gsem)

        def wait_gather(b):
            pltpu.make_async_copy(x0.at[pl.ds(0, k)], rows[b], gsem).wait()

        def issue_scatter(b):
            pltpu.async_copy(rows[b], acc.at[dscat[b]], ssem, add=True)

        def drain_scatter(b):
            pltpu.make_async_copy(x0.at[pl.ds(0, k)], rows[b], ssem).wait()

        def scale(b):
            rw = rows[b]
            wr = wbuf[b]

            def body(g, _):
                wv = wr[pl.ds(g * _L, _L)]
                for t in range(_L):
                    w16 = jnp.take_along_axis(
                        wv, jnp.full((_L,), t, jnp.int32), axis=0,
                        mode="promise_in_bounds")
                    i = g * _L + t
                    for fj in range(fch):
                        sl = pl.ds(fj * _L, _L)
                        rw[i, sl] = rw[i, sl] * w16
                return 0
            lax.fori_loop(0, g_per_k, body, 0)

        def step(ci, b, drain):
            nb = (b + 1) % ring
            b2 = (b + 2) % ring
            del nb
            wait_idx(b2)
            compute_w(b2)
            if drain:
                drain_scatter(b2)
            copy_dscat(b2)
            issue_gather(b2)
            wait_gather(b)
            issue_idx(ci + 3, b)
            scale(b)
            issue_scatter(b)

        issue_idx(jnp.int32(0), 0)
        wait_idx(0)
        compute_w(0)
        copy_dscat(0)
        issue_gather(0)
        issue_idx(jnp.int32(1), 1)
        wait_idx(1)
        compute_w(1)
        copy_dscat(1)
        issue_gather(1)
        issue_idx(jnp.int32(2), 2)
        step(jnp.int32(0), 0, drain=False)
        step(jnp.int32(1), 1, drain=True)

        def tri(j, _):
            ci = 2 + 3 * j
            step(ci, 2, drain=True)
            step(ci + 1, 0, drain=True)
            step(ci + 2, 1, drain=True)
            return 0
        lax.fori_loop(0, (n_chunks - 2) // 3, tri, 0)
        drain_scatter(1)
        wait_gather(2)
        wait_gather(0)
        wait_idx(0)

        plsc.subcore_barrier()
        for t in range(n_ob):
            rb = s * rows_out + t * k
            pltpu.sync_copy(acc.at[pl.ds(rb, k)], rw0)

            def ob(g, _):
                rv = rin_ep[pl.ds(t * k + g * _L, _L)]
                for u in range(_L):
                    r16 = jnp.take_along_axis(
                        rv, jnp.full((_L,), u, jnp.int32), axis=0,
                        mode="promise_in_bounds")
                    i = g * _L + u
                    for fj in range(fch):
                        sl = pl.ds(fj * _L, _L)
                        rw0[i, sl] = rw0[i, sl] * r16
                return 0
            lax.fori_loop(0, g_per_k, ob, 0)
            pltpu.sync_copy(rw0, out.at[c, pl.ds(rb, k)])

    return msg


def kernel(x, edge_index, edge_attrs):
    n, d = x.shape
    e = edge_index.shape[1]
    dh = d // 2
    k = 80
    n_pad = ((n + 255) // 256) * 256
    ids2 = jnp.concatenate(
        [edge_index[1], edge_index[0], jnp.zeros((256,), jnp.int32)])
    attrs = jnp.concatenate([edge_attrs, jnp.zeros((256,), jnp.float32)])
    r2 = _make_stats(e, n_pad)(ids2, attrs)
    nct = e // k
    pk = jnp.stack(
        [edge_index[1].reshape(nct, k), edge_index[0].reshape(nct, k),
         jax.lax.bitcast_convert_type(edge_attrs, jnp.int32).reshape(nct, k)],
        axis=1).reshape(-1)
    pk = jnp.concatenate([pk, jnp.zeros((4 * 3 * k,), jnp.int32)])
    o = _make_msg(n, e, n_pad, dh)(x[:, :dh], x[:, dh:], pk, r2)
    return jnp.concatenate([o[0, :n], o[1, :n]], axis=1)


# speedup vs baseline: 1.2792x; 1.0751x over previous
import functools

import jax
import jax.numpy as jnp
from jax import lax
from jax.experimental import pallas as pl
from jax.experimental.pallas import tpu as pltpu
from jax.experimental.pallas import tpu_sc as plsc

_L = 16
_NC = 2
_NS = 16


def _rsqrt_newton(s):
    bits = plsc.bitcast(s, jnp.int32)
    y = plsc.bitcast(jnp.full((_L,), 0x5F3759DF, jnp.int32) - (bits >> 1),
                     jnp.float32)
    for _ in range(3):
        y = y * (1.5 - 0.5 * s * y * y)
    return y


@functools.cache
def _make_stats(e, n_pad):
    ch = 2000
    per_tile = e // _NS
    n_chunks = per_tile // ch
    groups = ch // _L
    rpt = n_pad // _NS
    mesh = plsc.VectorSubcoreMesh(core_axis_name="c", subcore_axis_name="s",
                                  num_cores=_NC, num_subcores=_NS)

    @functools.partial(
        pl.kernel,
        out_type=jax.ShapeDtypeStruct((_NC, n_pad), jnp.float32),
        mesh=mesh,
        scratch_types=[
            pltpu.VMEM((ch,), jnp.int32),
            pltpu.VMEM((ch,), jnp.float32),
            pltpu.VMEM((n_pad,), jnp.float32),
            pltpu.VMEM((_NS, rpt), jnp.float32),
            pltpu.VMEM((rpt,), jnp.float32),
            pltpu.VMEM_SHARED((_NS, n_pad), jnp.float32),
        ],
        compiler_params=pltpu.CompilerParams(needs_layout_passes=False),
    )
    def stats(ids2, attrs, r_out, ids_v, at_v, table, colblk, rbuf, staging):
        c = lax.axis_index("c")
        s = lax.axis_index("s")

        def zero_body(i, _):
            table[pl.ds(i * _L, _L)] = jnp.zeros((_L,), jnp.float32)
            return 0
        lax.fori_loop(0, n_pad // _L, zero_body, 0)

        ebase = s * per_tile
        for chunk in range(n_chunks):
            base = ebase + chunk * ch
            pltpu.sync_copy(ids2.at[pl.ds(c * e + base, ch)], ids_v)
            pltpu.sync_copy(attrs.at[pl.ds(base, ch)], at_v)

            def upd(g, _):
                sl = pl.ds(g * _L, _L)
                plsc.addupdate_scatter(table, [ids_v[sl]], jnp.exp(at_v[sl]))
                return 0
            lax.fori_loop(0, groups, upd, 0)

        pltpu.sync_copy(table, staging.at[s])
        plsc.subcore_barrier()
        pltpu.sync_copy(staging.at[:, pl.ds(s * rpt, rpt)], colblk)

        def red(g, _):
            sl = pl.ds(g * _L, _L)
            acc = colblk[0, sl]
            for j in range(1, _NS):
                acc = acc + colblk[j, sl]
            rbuf[sl] = _rsqrt_newton(acc)
            return 0
        lax.fori_loop(0, rpt // _L, red, 0)
        pltpu.sync_copy(rbuf, r_out.at[c, pl.ds(s * rpt, rpt)])

    return stats


@functools.cache
def _make_msg(n, e, n_pad, dh):
    k = 80
    ring = 3
    per_tile = e // _NS
    n_chunks = per_tile // k
    g_per_k = k // _L
    fch = dh // _L
    rows_out = n_pad // _NS
    n_ob = rows_out // k
    mesh = plsc.VectorSubcoreMesh(core_axis_name="c", subcore_axis_name="s",
                                  num_cores=_NC, num_subcores=_NS)

    pk_t = [pltpu.VMEM((3 * k,), jnp.int32) for _ in range(ring)]
    idx_t = [pltpu.VMEM((k,), jnp.int32) for _ in range(ring)]
    val_t = [pltpu.VMEM((k,), jnp.float32) for _ in range(ring)]
    row_t = [pltpu.VMEM((k, dh), jnp.float32) for _ in range(ring)]

    @functools.partial(
        pl.kernel,
        out_type=jax.ShapeDtypeStruct((n, _NC * dh), jnp.float32),
        mesh=mesh,
        scratch_types=(
            [pltpu.VMEM((n_pad,), jnp.float32)]
            + [pltpu.VMEM((rows_out,), jnp.float32)]
            + pk_t
            + idx_t
            + val_t
            + row_t
            + [
                pltpu.VMEM_SHARED((n_pad, dh), jnp.float32),
                pltpu.SemaphoreType.DMA,
                pltpu.SemaphoreType.DMA,
                pltpu.SemaphoreType.DMA,
            ]
        ),
        compiler_params=pltpu.CompilerParams(needs_layout_passes=False),
    )
    def msg(x0, x1, pk, r2, out,
            rout_t, rin_ep,
            pk0, pk1, pk2, dsc0, dsc1, dsc2, wb0, wb1, wb2, rw0, rw1, rw2,
            acc, gsem, ssem, isem):
        pbuf = [pk0, pk1, pk2]
        dscat = [dsc0, dsc1, dsc2]
        wbuf = [wb0, wb1, wb2]
        rows = [rw0, rw1, rw2]
        c = lax.axis_index("c")
        s = lax.axis_index("s")
        pltpu.sync_copy(r2.at[1], rout_t)
        pltpu.sync_copy(r2.at[0, pl.ds(s * rows_out, rows_out)], rin_ep)

        def zb(i, _):
            for j in range(fch):
                rw0[i, pl.ds(j * _L, _L)] = jnp.zeros((_L,), jnp.float32)
            return 0
        lax.fori_loop(0, k, zb, 0)
        for t in range(n_ob):
            pltpu.sync_copy(rw0, acc.at[pl.ds(s * rows_out + t * k, k)])
        plsc.subcore_barrier()

        def issue_idx(ci, b):
            base = (s * n_chunks + ci) * (3 * k)
            pltpu.async_copy(pk.at[pl.ds(base, 3 * k)], pbuf[b], isem)

        def wait_idx(b):
            pltpu.make_async_copy(pk.at[pl.ds(0, 3 * k)], pbuf[b], isem).wait()

        def compute_w(b):
            for g in range(g_per_k):
                sg = pbuf[b][pl.ds(k + g * _L, _L)]
                av = plsc.bitcast(pbuf[b][pl.ds(2 * k + g * _L, _L)],
                                  jnp.float32)
                ro = plsc.load_gather(rout_t, [sg])
                wbuf[b][pl.ds(g * _L, _L)] = jnp.exp(av) * ro

        def copy_dscat(b):
            for g in range(g_per_k):
                sl = pl.ds(g * _L, _L)
                dscat[b][sl] = pbuf[b][sl]

        def issue_gather(b):
            idxr = pbuf[b].at[pl.ds(k, k)]

            @pl.when(c == 0)
            def _():
                pltpu.async_copy(x0.at[idxr], rows[b], gsem)

            @pl.when(c == 1)
            def _():
                pltpu.async_copy(x1.at[idxr], rows[b], gsem)

        def wait_gather(b):
            pltpu.make_async_copy(x0.at[pl.ds(0, k)], rows[b], gsem).wait()

        def issue_scatter(b):
            pltpu.async_copy(rows[b], acc.at[dscat[b]], ssem, add=True)

        def drain_scatter(b):
            pltpu.make_async_copy(x0.at[pl.ds(0, k)], rows[b], ssem).wait()

        def scale(b):
            rw = rows[b]
            wr = wbuf[b]

            def body(g, _):
                wv = wr[pl.ds(g * _L, _L)]
                for t in range(_L):
                    w16 = jnp.take_along_axis(
                        wv, jnp.full((_L,), t, jnp.int32), axis=0,
                        mode="promise_in_bounds")
                    i = g * _L + t
                    for fj in range(fch):
                        sl = pl.ds(fj * _L, _L)
                        rw[i, sl] = rw[i, sl] * w16
                return 0
            lax.fori_loop(0, g_per_k, body, 0)

        def step(ci, b, drain):
            nb = (b + 1) % ring
            b2 = (b + 2) % ring
            del nb
            wait_idx(b2)
            compute_w(b2)
            if drain:
                drain_scatter(b2)
            copy_dscat(b2)
            issue_gather(b2)
            wait_gather(b)
            issue_idx(ci + 3, b)
            scale(b)
            issue_scatter(b)

        issue_idx(jnp.int32(0), 0)
        wait_idx(0)
        compute_w(0)
        copy_dscat(0)
        issue_gather(0)
        issue_idx(jnp.int32(1), 1)
        wait_idx(1)
        compute_w(1)
        copy_dscat(1)
        issue_gather(1)
        issue_idx(jnp.int32(2), 2)
        step(jnp.int32(0), 0, drain=False)
        step(jnp.int32(1), 1, drain=True)

        def tri(j, _):
            ci = 2 + 3 * j
            step(ci, 2, drain=True)
            step(ci + 1, 0, drain=True)
            step(ci + 2, 1, drain=True)
            return 0
        lax.fori_loop(0, (n_chunks - 2) // 3, tri, 0)
        drain_scatter(1)
        wait_gather(2)
        wait_gather(0)
        wait_idx(0)

        plsc.subcore_barrier()

        def outcopy(nblk):
            for t in range(nblk):
                rb = s * rows_out + t * k
                pltpu.sync_copy(acc.at[pl.ds(rb, k)], rw0)

                def ob(g, _):
                    rv = rin_ep[pl.ds(t * k + g * _L, _L)]
                    for u in range(_L):
                        r16 = jnp.take_along_axis(
                            rv, jnp.full((_L,), u, jnp.int32), axis=0,
                            mode="promise_in_bounds")
                        i = g * _L + u
                        for fj in range(fch):
                            sl = pl.ds(fj * _L, _L)
                            rw0[i, sl] = rw0[i, sl] * r16
                    return 0
                lax.fori_loop(0, g_per_k, ob, 0)

                @pl.when(c == 0)
                def _():
                    pltpu.sync_copy(rw0, out.at[pl.ds(rb, k), pl.ds(0, dh)])

                @pl.when(c == 1)
                def _():
                    pltpu.sync_copy(rw0, out.at[pl.ds(rb, k), pl.ds(dh, dh)])

        full_tiles = (n // k) // n_ob

        @pl.when(s < full_tiles)
        def _():
            outcopy(n_ob)

        @pl.when(s >= full_tiles)
        def _():
            outcopy((n - full_tiles * rows_out) // k)

    return msg


def kernel(x, edge_index, edge_attrs):
    n, d = x.shape
    e = edge_index.shape[1]
    dh = d // 2
    k = 80
    n_pad = ((n + 255) // 256) * 256
    ids2 = jnp.concatenate([edge_index[1], edge_index[0]])
    r2 = _make_stats(e, n_pad)(ids2, edge_attrs)
    nct = e // k
    pk = jnp.stack(
        [edge_index[1].reshape(nct, k), edge_index[0].reshape(nct, k),
         jax.lax.bitcast_convert_type(edge_attrs, jnp.int32).reshape(nct, k)],
        axis=1).reshape(-1)
    pk = jnp.concatenate([pk, jnp.zeros((4 * 3 * k,), jnp.int32)])
    return _make_msg(n, e, n_pad, dh)(x[:, :dh], x[:, dh:], pk, r2)
